# baseline (speedup 1.0000x reference)
import functools

import jax
import jax.numpy as jnp
from jax import lax
from jax.experimental import pallas as pl
from jax.experimental.pallas import tpu as pltpu

_B = 16384
_D = 64
_V = 1000000
_W = 512
_NWIN = -(-_V // _W)
_K = 8
_SCALE = 0.5 * 0.5 / _B


def _win_body(starts_ref, xl_hbm, c_ref, out_ref, xl_ref, sem):
    t = pl.program_id(0)

    @pl.when(t == 0)
    def _():
        cp = pltpu.make_async_copy(xl_hbm, xl_ref, sem)
        cp.start()
        cp.wait()

    c = c_ref[...]
    nwin = jnp.sum(c * c, axis=0, keepdims=True)
    lanef = (t * _W
             + lax.broadcasted_iota(jnp.int32, (1, _W), 1)).astype(jnp.float32)

    gs = starts_ref[t]
    ge = starts_ref[t + 1]
    nch = (ge - gs + _K - 1) // _K

    def chunk(i, acc):
        rs = gs + i * _K
        rs_c = jnp.minimum(rs, _B - _K)
        xs = xl_ref[pl.ds(rs_c, _K), 0:_D]
        labf = xl_ref[pl.ds(rs_c, _K), _D:_D + 1]
        pos = rs_c + lax.broadcasted_iota(jnp.int32, (_K, 1), 0)
        valid = (pos >= rs) & (pos < ge)
        oh = (labf == lanef) & valid
        g = jnp.dot(xs, c, preferred_element_type=jnp.float32)
        return acc + jnp.sum(jnp.where(oh, nwin - 2.0 * g, 0.0))

    acc = lax.fori_loop(0, nch, chunk, jnp.float32(0.0))

    @pl.when(t == 0)
    def _():
        xall = xl_ref[:, 0:_D]
        out_ref[...] = jnp.full((1, 8, 128), acc + jnp.sum(xall * xall),
                                jnp.float32)

    @pl.when(t != 0)
    def _():
        out_ref[...] = jnp.full((1, 8, 128), acc, jnp.float32)


@jax.jit
def _center_loss(x, labels_i32, centers):
    order = jnp.argsort(labels_i32)
    labs = labels_i32[order]
    xs = x[order]
    xl = jnp.concatenate(
        [xs, labs.astype(jnp.float32)[:, None],
         jnp.zeros((_B, 128 - _D - 1), jnp.float32)], axis=1)
    bounds = jnp.arange(_NWIN + 1, dtype=jnp.int32) * _W
    starts = jnp.searchsorted(labs, bounds).astype(jnp.int32)

    partials = pl.pallas_call(
        _win_body,
        grid_spec=pltpu.PrefetchScalarGridSpec(
            num_scalar_prefetch=1,
            grid=(_NWIN,),
            in_specs=[
                pl.BlockSpec(memory_space=pltpu.MemorySpace.HBM),
                pl.BlockSpec((_D, _W), lambda t, s: (0, t)),
            ],
            out_specs=pl.BlockSpec((1, 8, 128), lambda t, s: (t, 0, 0)),
            scratch_shapes=[
                pltpu.VMEM((_B, 128), jnp.float32),
                pltpu.SemaphoreType.DMA,
            ],
        ),
        out_shape=jax.ShapeDtypeStruct((_NWIN, 8, 128), jnp.float32),
    )(starts, xl, centers.T)
    return _SCALE * jnp.sum(partials[:, 0, 0])


def kernel(x, labels, centers):
    return _center_loss(x, labels.astype(jnp.int32), centers)


# speedup vs baseline: 2.0049x; 2.0049x over previous
import functools

import jax
import jax.numpy as jnp
from jax import lax
from jax.experimental import pallas as pl
from jax.experimental.pallas import tpu as pltpu

_B = 16384
_D = 64
_V = 1000000
_W = 2048
_NWIN = -(-_V // _W)
_K = 8
_SCALE = 0.5 * 0.5 / _B


def _win_body(starts_ref, xl_hbm, c_ref, out_ref, xl_ref, sem):
    t = pl.program_id(0)

    @pl.when(t == 0)
    def _():
        cp = pltpu.make_async_copy(xl_hbm, xl_ref, sem)
        cp.start()
        cp.wait()

    c = c_ref[...]
    nwin = jnp.sum(c * c, axis=0, keepdims=True)
    lanef = (t * _W
             + lax.broadcasted_iota(jnp.int32, (1, _W), 1)).astype(jnp.float32)

    gs = starts_ref[t]
    ge = starts_ref[t + 1]
    nch = (ge - gs + _K - 1) // _K

    def chunk(i, acc):
        rs = gs + i * _K
        rs_c = jnp.minimum(rs, _B - _K)
        xs = xl_ref[pl.ds(rs_c, _K), 0:_D]
        labf = xl_ref[pl.ds(rs_c, _K), _D:_D + 1]
        pos = rs_c + lax.broadcasted_iota(jnp.int32, (_K, 1), 0)
        valid = (pos >= rs) & (pos < ge)
        oh = (labf == lanef) & valid
        g = jnp.dot(xs, c, preferred_element_type=jnp.float32)
        return acc + jnp.sum(jnp.where(oh, nwin - 2.0 * g, 0.0))

    acc = lax.fori_loop(0, nch, chunk, jnp.float32(0.0))

    @pl.when(t == 0)
    def _():
        xall = xl_ref[:, 0:_D]
        out_ref[...] = jnp.full((1, 8, 128), acc + jnp.sum(xall * xall),
                                jnp.float32)

    @pl.when(t != 0)
    def _():
        out_ref[...] = jnp.full((1, 8, 128), acc, jnp.float32)


@jax.jit
def _center_loss(x, labels_i32, centers):
    order = jnp.argsort(labels_i32)
    labs = labels_i32[order]
    xs = x[order]
    xl = jnp.concatenate(
        [xs, labs.astype(jnp.float32)[:, None],
         jnp.zeros((_B, 128 - _D - 1), jnp.float32)], axis=1)
    bounds = jnp.arange(_NWIN + 1, dtype=jnp.int32) * _W
    starts = jnp.searchsorted(labs, bounds).astype(jnp.int32)

    partials = pl.pallas_call(
        _win_body,
        grid_spec=pltpu.PrefetchScalarGridSpec(
            num_scalar_prefetch=1,
            grid=(_NWIN,),
            in_specs=[
                pl.BlockSpec(memory_space=pltpu.MemorySpace.HBM),
                pl.BlockSpec((_D, _W), lambda t, s: (0, t)),
            ],
            out_specs=pl.BlockSpec((1, 8, 128), lambda t, s: (t, 0, 0)),
            scratch_shapes=[
                pltpu.VMEM((_B, 128), jnp.float32),
                pltpu.SemaphoreType.DMA,
            ],
        ),
        out_shape=jax.ShapeDtypeStruct((_NWIN, 8, 128), jnp.float32),
    )(starts, xl, centers.T)
    return _SCALE * jnp.sum(partials[:, 0, 0])


def kernel(x, labels, centers):
    return _center_loss(x, labels.astype(jnp.int32), centers)
